# baseline (speedup 1.0000x reference)
import functools

import jax
import jax.numpy as jnp
from jax import lax
from jax.experimental import pallas as pl
from jax.experimental.pallas import tpu as pltpu
from jax.experimental.pallas import tpu_sc as plsc

_RC = 8


@functools.cache
def _make_sc(R, D, S):
    NW = 32
    srange = S // NW
    B = R // S
    CW = _RC * D

    @functools.partial(
        pl.kernel,
        out_type=jax.ShapeDtypeStruct((R * D,), jnp.float32),
        mesh=plsc.VectorSubcoreMesh(core_axis_name="c", subcore_axis_name="s"),
        scratch_types=[
            pltpu.VMEM((CW,), jnp.float32),
            pltpu.VMEM((CW,), jnp.float32),
        ],
    )
    def k(x_flat, pos_flat, out_flat, xbuf, pbuf):
        wid = lax.axis_index("s") * 2 + lax.axis_index("c")
        s0 = wid * srange
        for c in range(srange // _RC):
            base_s = s0 + c * _RC
            pltpu.sync_copy(pos_flat.at[pl.ds(base_s * D, CW)], pbuf)
            for b in range(B):
                w0 = (b * S + base_s) * D
                pltpu.sync_copy(x_flat.at[pl.ds(w0, CW)], xbuf)

                @plsc.parallel_loop(0, CW, 16, unroll=8)
                def _(i):
                    plsc.addupdate(xbuf.at[pl.ds(i, 16)], pbuf[pl.ds(i, 16)])

                pltpu.sync_copy(xbuf, out_flat.at[pl.ds(w0, CW)])

    return k


def kernel(x, pos_table):
    B, S, D = x.shape
    out = _make_sc(B * S, D, S)(x.reshape(B * S * D), pos_table.reshape(-1))
    return out.reshape(B, S, D)


# speedup vs baseline: 1.2095x; 1.2095x over previous
import functools

import jax
import jax.numpy as jnp
from jax import lax
from jax.experimental import pallas as pl
from jax.experimental.pallas import tpu as pltpu
from jax.experimental.pallas import tpu_sc as plsc

_RC = 16


@functools.cache
def _make_sc(R, D, S):
    NW = 32
    srange = S // NW
    B = R // S
    CW = _RC * D
    NC_CHUNKS = srange // _RC
    NT = NC_CHUNKS * B

    @functools.partial(
        pl.kernel,
        out_type=jax.ShapeDtypeStruct((R * D,), jnp.float32),
        mesh=plsc.VectorSubcoreMesh(core_axis_name="c", subcore_axis_name="s"),
        scratch_types=[
            pltpu.VMEM((CW,), jnp.float32),
            pltpu.VMEM((CW,), jnp.float32),
            pltpu.VMEM((CW,), jnp.float32),
            pltpu.SemaphoreType.DMA,
            pltpu.SemaphoreType.DMA,
            pltpu.SemaphoreType.DMA,
            pltpu.SemaphoreType.DMA,
        ],
    )
    def k(x_flat, pos_flat, out_flat, xb0, xb1, pbuf, si0, si1, so0, so1):
        wid = lax.axis_index("s") * 2 + lax.axis_index("c")
        s0 = wid * srange
        xb = (xb0, xb1)
        sin = (si0, si1)
        sout = (so0, so1)

        def word0(t):
            c, b = divmod(t, B)
            return (b * S + s0 + c * _RC) * D

        d_in = [None, None]
        d_out = [None, None]
        d_in[0] = pltpu.async_copy(x_flat.at[pl.ds(word0(0), CW)], xb[0], sin[0])
        for t in range(NT):
            p = t % 2
            if t + 1 < NT:
                if d_out[(t + 1) % 2] is not None:
                    d_out[(t + 1) % 2].wait()
                d_in[(t + 1) % 2] = pltpu.async_copy(
                    x_flat.at[pl.ds(word0(t + 1), CW)], xb[(t + 1) % 2], sin[(t + 1) % 2]
                )
            if t % B == 0:
                base_s = s0 + (t // B) * _RC
                pltpu.sync_copy(pos_flat.at[pl.ds(base_s * D, CW)], pbuf)
            d_in[p].wait()

            @plsc.parallel_loop(0, CW, 16, unroll=8)
            def _(i):
                plsc.addupdate(xb[p].at[pl.ds(i, 16)], pbuf[pl.ds(i, 16)])

            d_out[p] = pltpu.async_copy(xb[p], out_flat.at[pl.ds(word0(t), CW)], sout[p])
        d_out[(NT - 1) % 2].wait()
        d_out[NT % 2].wait()

    return k


def kernel(x, pos_table):
    B, S, D = x.shape
    out = _make_sc(B * S, D, S)(x.reshape(B * S * D), pos_table.reshape(-1))
    return out.reshape(B, S, D)


# speedup vs baseline: 5.8261x; 4.8168x over previous
import jax
import jax.numpy as jnp
from jax.experimental import pallas as pl

_BS = 512
_BB = 2


def _add_kernel(x_ref, pos_ref, o_ref):
    o_ref[...] = x_ref[...] + pos_ref[...]


def kernel(x, pos_table):
    B, S, D = x.shape
    grid = (S // _BS, B // _BB)
    return pl.pallas_call(
        _add_kernel,
        grid=grid,
        in_specs=[
            pl.BlockSpec((_BB, _BS, D), lambda s, b: (b, s, 0)),
            pl.BlockSpec((_BS, D), lambda s, b: (s, 0)),
        ],
        out_specs=pl.BlockSpec((_BB, _BS, D), lambda s, b: (b, s, 0)),
        out_shape=jax.ShapeDtypeStruct((B, S, D), x.dtype),
    )(x, pos_table)


# speedup vs baseline: 5.8488x; 1.0039x over previous
import jax
import jax.numpy as jnp
from jax.experimental import pallas as pl

_BS = 1024


def _add_kernel(x_ref, pos_ref, o_ref):
    o_ref[...] = x_ref[...] + pos_ref[...]


def kernel(x, pos_table):
    B, S, D = x.shape
    grid = (S // _BS, B)
    return pl.pallas_call(
        _add_kernel,
        grid=grid,
        in_specs=[
            pl.BlockSpec((1, _BS, D), lambda s, b: (b, s, 0)),
            pl.BlockSpec((_BS, D), lambda s, b: (s, 0)),
        ],
        out_specs=pl.BlockSpec((1, _BS, D), lambda s, b: (b, s, 0)),
        out_shape=jax.ShapeDtypeStruct((B, S, D), x.dtype),
    )(x, pos_table)
